# Initial kernel scaffold; baseline (speedup 1.0000x reference)
import jax
import jax.numpy as jnp
from jax.experimental import pallas as pl


def kernel(features, indices, freq):
    raise NotImplementedError("write your pallas kernel here")


# speedup vs baseline: 1.1618x; 1.1618x over previous
import functools

import jax
import jax.numpy as jnp
from jax import lax
from jax.experimental import pallas as pl
from jax.experimental.pallas import tpu as pltpu
from jax.experimental.pallas import tpu_sc as plsc

_DIM = 96
_GRID = 64
_D = 3 * _DIM
_N = 32768
_T = 128


def _build_tables(freq_rep):
    def body(freq_ref, cos_ref, sin_ref):
        fr = freq_ref[...]
        s = lax.broadcasted_iota(jnp.int32, (_GRID, _DIM), 0).astype(jnp.float32)
        ang = s * fr
        col = lax.broadcasted_iota(jnp.int32, (_GRID, _DIM), 1)
        sgn = jnp.where((col & 1) == 0, -1.0, 1.0)
        cos_ref[...] = jnp.cos(ang)
        sin_ref[...] = sgn * jnp.sin(ang)

    return pl.pallas_call(
        body,
        out_shape=(jax.ShapeDtypeStruct((_GRID, _DIM), jnp.float32),
                   jax.ShapeDtypeStruct((_GRID, _DIM), jnp.float32)),
    )(freq_rep)


def _rope_sc(feat_flat, idx_flat, cos_flat, sin_flat):
    info = plsc.get_sparse_core_info()
    nc = info.num_cores
    nw = nc * info.num_subcores
    per_w = _N // nw
    nblk = per_w // _T
    mesh = plsc.VectorSubcoreMesh(core_axis_name="c", subcore_axis_name="s")

    @functools.partial(
        pl.kernel,
        mesh=mesh,
        compiler_params=pltpu.CompilerParams(needs_layout_passes=False),
        out_type=jax.ShapeDtypeStruct((_N * _D,), jnp.float32),
        scratch_types=[
            pltpu.VMEM((_T * _D,), jnp.float32),
            pltpu.VMEM((_T * _D,), jnp.float32),
            pltpu.VMEM((_T * 4,), jnp.int32),
            pltpu.VMEM((_GRID * _DIM,), jnp.float32),
            pltpu.VMEM((_GRID * _DIM,), jnp.float32),
        ],
    )
    def k(feat_hbm, idx_hbm, cos_hbm, sin_hbm, out_hbm,
          fbuf, obuf, cbuf, cosb, sinb):
        wid = lax.axis_index("s") * nc + lax.axis_index("c")
        pltpu.sync_copy(cos_hbm, cosb)
        pltpu.sync_copy(sin_hbm, sinb)
        swap = lax.iota(jnp.int32, 16) ^ 1

        def blk(b, carry):
            base = pl.multiple_of((wid * nblk + b) * _T, _T)
            pltpu.sync_copy(feat_hbm.at[pl.ds(base * _D, _T * _D)], fbuf)
            pltpu.sync_copy(idx_hbm.at[pl.ds(base * 4, _T * 4)], cbuf)

            def tok4(r, c2):
                iv = cbuf[pl.ds(r * 16, 16)]
                for t in range(4):
                    i = r * 4 + t
                    for a in range(3):
                        s = iv[4 * t + 3 - a]
                        srow = s * _DIM
                        for j in range(_DIM // 16):
                            off = i * _D + a * _DIM + j * 16
                            f = fbuf[pl.ds(off, 16)]
                            fs = plsc.load_gather(fbuf, [off + swap])
                            cv = cosb[pl.ds(srow + j * 16, 16)]
                            sv = sinb[pl.ds(srow + j * 16, 16)]
                            obuf[pl.ds(off, 16)] = f * cv + fs * sv
                return c2

            lax.fori_loop(0, _T // 4, tok4, 0)
            pltpu.sync_copy(obuf, out_hbm.at[pl.ds(base * _D, _T * _D)])
            return carry

        lax.fori_loop(0, nblk, blk, 0)

    return k(feat_flat, idx_flat, cos_flat, sin_flat)


def kernel(features, indices, freq):
    freq_rep = jnp.repeat(freq.astype(jnp.float32), 2).reshape(1, _DIM)
    cos_t, sin_t = _build_tables(freq_rep)
    out = _rope_sc(features.reshape(-1), indices.astype(jnp.int32).reshape(-1),
                   cos_t.reshape(-1), sin_t.reshape(-1))
    return out.reshape(_N, _D)


# speedup vs baseline: 1.1950x; 1.0286x over previous
import functools

import jax
import jax.numpy as jnp
from jax import lax
from jax.experimental import pallas as pl
from jax.experimental.pallas import tpu as pltpu
from jax.experimental.pallas import tpu_sc as plsc

_DIM = 96
_GRID = 64
_D = 3 * _DIM
_N = 32768
_T = 128


def _build_tables(freq_rep):
    def body(freq_ref, cos_ref, sin_ref):
        fr = freq_ref[...]
        s = lax.broadcasted_iota(jnp.int32, (_GRID, _DIM), 0).astype(jnp.float32)
        ang = s * fr
        col = lax.broadcasted_iota(jnp.int32, (_GRID, _DIM), 1)
        sgn = jnp.where((col & 1) == 0, -1.0, 1.0)
        cos_ref[...] = jnp.cos(ang)
        sin_ref[...] = sgn * jnp.sin(ang)

    return pl.pallas_call(
        body,
        out_shape=(jax.ShapeDtypeStruct((_GRID, _DIM), jnp.float32),
                   jax.ShapeDtypeStruct((_GRID, _DIM), jnp.float32)),
    )(freq_rep)


def _rope_sc(feat_flat, idx_flat, cos_flat, sin_flat):
    info = plsc.get_sparse_core_info()
    nc = info.num_cores
    nw = nc * info.num_subcores
    per_w = _N // nw
    nblk = per_w // _T
    mesh = plsc.VectorSubcoreMesh(core_axis_name="c", subcore_axis_name="s")

    @functools.partial(
        pl.kernel,
        mesh=mesh,
        compiler_params=pltpu.CompilerParams(needs_layout_passes=False),
        out_type=jax.ShapeDtypeStruct((_N * _D,), jnp.float32),
        scratch_types=[
            pltpu.VMEM((_T * _D,), jnp.float32),
            pltpu.VMEM((_T * _D,), jnp.float32),
            pltpu.VMEM((_T * 4,), jnp.int32),
            pltpu.VMEM((_GRID * _DIM,), jnp.float32),
            pltpu.VMEM((_GRID * _DIM,), jnp.float32),
        ],
    )
    def k(feat_hbm, idx_hbm, cos_hbm, sin_hbm, out_hbm,
          fbuf, obuf, cbuf, cosb, sinb):
        wid = lax.axis_index("s") * nc + lax.axis_index("c")
        pltpu.sync_copy(cos_hbm, cosb)
        pltpu.sync_copy(sin_hbm, sinb)
        swap = lax.iota(jnp.int32, 16) ^ 1

        def blk(b, carry):
            base = pl.multiple_of((wid * nblk + b) * _T, _T)
            pltpu.sync_copy(feat_hbm.at[pl.ds(base * _D, _T * _D)], fbuf)
            pltpu.sync_copy(idx_hbm.at[pl.ds(base * 4, _T * 4)], cbuf)

            def tok4(r, c2):
                iv = cbuf[pl.ds(r * 16, 16)]
                for t in range(4):
                    i = r * 4 + t
                    for a in range(3):
                        s = iv[4 * t + 3 - a]
                        srow = s * _DIM
                        for j in range(_DIM // 16):
                            off = i * _D + a * _DIM + j * 16
                            f = fbuf[pl.ds(off, 16)]
                            fs = jnp.take_along_axis(
                                f, swap, axis=0, mode="promise_in_bounds")
                            cv = cosb[pl.ds(srow + j * 16, 16)]
                            sv = sinb[pl.ds(srow + j * 16, 16)]
                            obuf[pl.ds(off, 16)] = f * cv + fs * sv
                return c2

            lax.fori_loop(0, _T // 4, tok4, 0)
            pltpu.sync_copy(obuf, out_hbm.at[pl.ds(base * _D, _T * _D)])
            return carry

        lax.fori_loop(0, nblk, blk, 0)

    return k(feat_flat, idx_flat, cos_flat, sin_flat)


def kernel(features, indices, freq):
    freq_rep = jnp.repeat(freq.astype(jnp.float32), 2).reshape(1, _DIM)
    cos_t, sin_t = _build_tables(freq_rep)
    out = _rope_sc(features.reshape(-1), indices.astype(jnp.int32).reshape(-1),
                   cos_t.reshape(-1), sin_t.reshape(-1))
    return out.reshape(_N, _D)


# speedup vs baseline: 1.4779x; 1.2367x over previous
import functools

import jax
import jax.numpy as jnp
from jax import lax
from jax.experimental import pallas as pl
from jax.experimental.pallas import tpu as pltpu
from jax.experimental.pallas import tpu_sc as plsc

_DIM = 96
_GRID = 64
_D = 3 * _DIM
_N = 32768
_T = 128


def _build_tables(freq_rep):
    def body(freq_ref, cos_ref, sin_ref):
        fr = freq_ref[...]
        s = lax.broadcasted_iota(jnp.int32, (_GRID, _DIM), 0).astype(jnp.float32)
        ang = s * fr
        col = lax.broadcasted_iota(jnp.int32, (_GRID, _DIM), 1)
        sgn = jnp.where((col & 1) == 0, -1.0, 1.0)
        cos_ref[...] = jnp.cos(ang)
        sin_ref[...] = sgn * jnp.sin(ang)

    return pl.pallas_call(
        body,
        out_shape=(jax.ShapeDtypeStruct((_GRID, _DIM), jnp.float32),
                   jax.ShapeDtypeStruct((_GRID, _DIM), jnp.float32)),
    )(freq_rep)


def _rope_sc(feat_flat, idx_flat, cos_flat, sin_flat):
    info = plsc.get_sparse_core_info()
    nc = info.num_cores
    nw = nc * info.num_subcores
    per_w = _N // nw
    nblk = per_w // _T
    mesh = plsc.VectorSubcoreMesh(core_axis_name="c", subcore_axis_name="s")

    @functools.partial(
        pl.kernel,
        mesh=mesh,
        compiler_params=pltpu.CompilerParams(needs_layout_passes=False),
        out_type=jax.ShapeDtypeStruct((_N, _D), jnp.float32),
        scratch_types=[
            pltpu.VMEM((_T, _D), jnp.float32),
            pltpu.VMEM((_T, _D), jnp.float32),
            pltpu.VMEM((_T * 4,), jnp.int32),
            pltpu.VMEM((_GRID * _DIM,), jnp.float32),
            pltpu.VMEM((_GRID * _DIM,), jnp.float32),
        ],
    )
    def k(feat_hbm, idx_hbm, cos_hbm, sin_hbm, out_hbm,
          fbuf, obuf, cbuf, cosb, sinb):
        wid = lax.axis_index("s") * nc + lax.axis_index("c")
        pltpu.sync_copy(cos_hbm, cosb)
        pltpu.sync_copy(sin_hbm, sinb)
        swap = lax.iota(jnp.int32, 16) ^ 1

        def blk(b, carry):
            base = pl.multiple_of((wid * nblk + b) * _T, _T)
            pltpu.sync_copy(feat_hbm.at[pl.ds(base, _T)], fbuf)
            pltpu.sync_copy(idx_hbm.at[pl.ds(base * 4, _T * 4)], cbuf)

            def tok4(r, c2):
                iv = cbuf[pl.ds(r * 16, 16)]
                for t in range(4):
                    i = r * 4 + t
                    for a in range(3):
                        s = iv[4 * t + 3 - a]
                        srow = s * _DIM
                        for j in range(_DIM // 16):
                            col = a * _DIM + j * 16
                            f = fbuf[i, pl.ds(col, 16)]
                            fs = jnp.take_along_axis(
                                f, swap, axis=0, mode="promise_in_bounds")
                            cv = cosb[pl.ds(srow + j * 16, 16)]
                            sv = sinb[pl.ds(srow + j * 16, 16)]
                            obuf[i, pl.ds(col, 16)] = f * cv + fs * sv
                return c2

            lax.fori_loop(0, _T // 4, tok4, 0)
            pltpu.sync_copy(obuf, out_hbm.at[pl.ds(base, _T)])
            return carry

        lax.fori_loop(0, nblk, blk, 0)

    return k(feat_flat, idx_flat, cos_flat, sin_flat)


def kernel(features, indices, freq):
    freq_rep = jnp.repeat(freq.astype(jnp.float32), 2).reshape(1, _DIM)
    cos_t, sin_t = _build_tables(freq_rep)
    return _rope_sc(features, indices.astype(jnp.int32).reshape(-1),
                    cos_t.reshape(-1), sin_t.reshape(-1))


# speedup vs baseline: 2.0875x; 1.4124x over previous
import functools

import jax
import jax.numpy as jnp
from jax import lax
from jax.experimental import pallas as pl
from jax.experimental.pallas import tpu as pltpu
from jax.experimental.pallas import tpu_sc as plsc

_DIM = 96
_GRID = 64
_D = 3 * _DIM
_N = 32768
_T = 128


def _build_tables(freq_rep):
    def body(freq_ref, cos_ref, sin_ref):
        fr = freq_ref[...]
        s = lax.broadcasted_iota(jnp.int32, (_GRID, _DIM), 0).astype(jnp.float32)
        ang = s * fr
        col = lax.broadcasted_iota(jnp.int32, (_GRID, _DIM), 1)
        sgn = jnp.where((col & 1) == 0, -1.0, 1.0)
        cos_ref[...] = jnp.cos(ang)
        sin_ref[...] = sgn * jnp.sin(ang)

    return pl.pallas_call(
        body,
        out_shape=(jax.ShapeDtypeStruct((_GRID, _DIM), jnp.float32),
                   jax.ShapeDtypeStruct((_GRID, _DIM), jnp.float32)),
    )(freq_rep)


def _rope_sc(feat_flat, idx_flat, cos_flat, sin_flat):
    info = plsc.get_sparse_core_info()
    nc = info.num_cores
    nw = nc * info.num_subcores
    per_w = _N // nw
    nblk = per_w // _T
    mesh = plsc.VectorSubcoreMesh(core_axis_name="c", subcore_axis_name="s")

    @functools.partial(
        pl.kernel,
        mesh=mesh,
        compiler_params=pltpu.CompilerParams(needs_layout_passes=False),
        out_type=jax.ShapeDtypeStruct((_N, _D), jnp.float32),
        scratch_types=[
            pltpu.VMEM((_T, _D), jnp.float32),
            pltpu.VMEM((_T, _D), jnp.float32),
            pltpu.VMEM((_T * 4,), jnp.int32),
            pltpu.VMEM((_GRID * _DIM,), jnp.float32),
            pltpu.VMEM((_GRID * _DIM,), jnp.float32),
        ],
    )
    def k(feat_hbm, idx_hbm, cos_hbm, sin_hbm, out_hbm,
          fbuf, obuf, cbuf, cosb, sinb):
        wid = lax.axis_index("s") * nc + lax.axis_index("c")
        pltpu.sync_copy(cos_hbm, cosb)
        pltpu.sync_copy(sin_hbm, sinb)
        swap = lax.iota(jnp.int32, 16) ^ 1

        def blk(b, carry):
            base = pl.multiple_of((wid * nblk + b) * _T, _T)
            pltpu.sync_copy(feat_hbm.at[pl.ds(base, _T)], fbuf)
            pltpu.sync_copy(idx_hbm.at[pl.ds(base * 4, _T * 4)], cbuf)

            @plsc.parallel_loop(0, _T // 4, unroll=2)
            def tok4(r):
                iv = cbuf[pl.ds(r * 16, 16)]
                for t in range(4):
                    i = r * 4 + t
                    for a in range(3):
                        s = iv[4 * t + 3 - a]
                        srow = s * _DIM
                        nj = _DIM // 16
                        fv = [fbuf[i, pl.ds(a * _DIM + j * 16, 16)]
                              for j in range(nj)]
                        cv = [cosb[pl.ds(srow + j * 16, 16)]
                              for j in range(nj)]
                        sv = [sinb[pl.ds(srow + j * 16, 16)]
                              for j in range(nj)]
                        for j in range(nj):
                            fs = jnp.take_along_axis(
                                fv[j], swap, axis=0, mode="promise_in_bounds")
                            obuf[i, pl.ds(a * _DIM + j * 16, 16)] = (
                                fv[j] * cv[j] + fs * sv[j])
            pltpu.sync_copy(obuf, out_hbm.at[pl.ds(base, _T)])
            return carry

        lax.fori_loop(0, nblk, blk, 0)

    return k(feat_flat, idx_flat, cos_flat, sin_flat)


def kernel(features, indices, freq):
    freq_rep = jnp.repeat(freq.astype(jnp.float32), 2).reshape(1, _DIM)
    cos_t, sin_t = _build_tables(freq_rep)
    return _rope_sc(features, indices.astype(jnp.int32).reshape(-1),
                    cos_t.reshape(-1), sin_t.reshape(-1))


# speedup vs baseline: 2.5245x; 1.2094x over previous
import functools

import jax
import jax.numpy as jnp
from jax import lax
from jax.experimental import pallas as pl
from jax.experimental.pallas import tpu as pltpu
from jax.experimental.pallas import tpu_sc as plsc

_DIM = 96
_GRID = 64
_D = 3 * _DIM
_N = 32768
_T = 64


def _build_tables(freq_rep):
    def body(freq_ref, cos_ref, sin_ref):
        fr = freq_ref[...]
        s = lax.broadcasted_iota(jnp.int32, (_GRID, _DIM), 0).astype(jnp.float32)
        ang = s * fr
        col = lax.broadcasted_iota(jnp.int32, (_GRID, _DIM), 1)
        sgn = jnp.where((col & 1) == 0, -1.0, 1.0)
        cos_ref[...] = jnp.cos(ang)
        sin_ref[...] = sgn * jnp.sin(ang)

    return pl.pallas_call(
        body,
        out_shape=(jax.ShapeDtypeStruct((_GRID, _DIM), jnp.float32),
                   jax.ShapeDtypeStruct((_GRID, _DIM), jnp.float32)),
    )(freq_rep)


def _rope_sc(feat_flat, idx_flat, cos_flat, sin_flat):
    info = plsc.get_sparse_core_info()
    nc = info.num_cores
    nw = nc * info.num_subcores
    per_w = _N // nw
    nblk = per_w // _T
    mesh = plsc.VectorSubcoreMesh(core_axis_name="c", subcore_axis_name="s")

    @functools.partial(
        pl.kernel,
        mesh=mesh,
        compiler_params=pltpu.CompilerParams(needs_layout_passes=False),
        out_type=jax.ShapeDtypeStruct((_N, _D), jnp.float32),
        scratch_types=[
            pltpu.VMEM((_T, _D), jnp.float32),
            pltpu.VMEM((_T, _D), jnp.float32),
            pltpu.VMEM((_T, _D), jnp.float32),
            pltpu.VMEM((_T, _D), jnp.float32),
            pltpu.VMEM((_N // 32 * 4,), jnp.int32),
            pltpu.VMEM((_GRID * _DIM,), jnp.float32),
            pltpu.VMEM((_GRID * _DIM,), jnp.float32),
            pltpu.SemaphoreType.DMA,
            pltpu.SemaphoreType.DMA,
            pltpu.SemaphoreType.DMA,
            pltpu.SemaphoreType.DMA,
        ],
    )
    def k(feat_hbm, idx_hbm, cos_hbm, sin_hbm, out_hbm,
          fbuf0, fbuf1, obuf0, obuf1, cbuf, cosb, sinb,
          sem_i0, sem_i1, sem_o0, sem_o1):
        wid = lax.axis_index("s") * nc + lax.axis_index("c")
        pltpu.sync_copy(cos_hbm, cosb)
        pltpu.sync_copy(sin_hbm, sinb)
        pltpu.sync_copy(idx_hbm.at[pl.ds(wid * (per_w * 4), per_w * 4)], cbuf)
        swap = lax.iota(jnp.int32, 16) ^ 1
        fbufs, obufs = (fbuf0, fbuf1), (obuf0, obuf1)
        sems_i, sems_o = (sem_i0, sem_i1), (sem_o0, sem_o1)

        def feat_rows(b):
            base = pl.multiple_of(wid * per_w + b * _T, _T)
            return feat_hbm.at[pl.ds(base, _T)]

        def out_rows(b):
            base = pl.multiple_of(wid * per_w + b * _T, _T)
            return out_hbm.at[pl.ds(base, _T)]

        def compute(b, fbuf, obuf):
            @plsc.parallel_loop(0, _T // 4, unroll=2)
            def tok4(r):
                g = b * (_T // 4) + r
                iv = cbuf[pl.ds(g * 16, 16)]
                for t in range(4):
                    i = r * 4 + t
                    for a in range(3):
                        s = iv[4 * t + 3 - a]
                        srow = s * _DIM
                        nj = _DIM // 16
                        fv = [fbuf[i, pl.ds(a * _DIM + j * 16, 16)]
                              for j in range(nj)]
                        cv = [cosb[pl.ds(srow + j * 16, 16)]
                              for j in range(nj)]
                        sv = [sinb[pl.ds(srow + j * 16, 16)]
                              for j in range(nj)]
                        for j in range(nj):
                            fs = jnp.take_along_axis(
                                fv[j], swap, axis=0, mode="promise_in_bounds")
                            obuf[i, pl.ds(a * _DIM + j * 16, 16)] = (
                                fv[j] * cv[j] + fs * sv[j])

        pltpu.async_copy(feat_rows(0), fbuf0, sem_i0)
        pltpu.async_copy(feat_rows(1), fbuf1, sem_i1)

        def pair(p, carry):
            for sl in range(2):
                b = p * 2 + sl
                pltpu.make_async_copy(feat_rows(0), fbufs[sl], sems_i[sl]).wait()

                @pl.when(p > 0)
                def _():
                    pltpu.make_async_copy(
                        obufs[sl], out_rows(0), sems_o[sl]).wait()

                compute(b, fbufs[sl], obufs[sl])
                pltpu.async_copy(obufs[sl], out_rows(b), sems_o[sl])

                @pl.when(b + 2 < nblk)
                def _():
                    pltpu.async_copy(feat_rows(b + 2), fbufs[sl], sems_i[sl])
            return carry

        lax.fori_loop(0, nblk // 2, pair, 0)
        pltpu.make_async_copy(obuf0, out_rows(0), sem_o0).wait()
        pltpu.make_async_copy(obuf1, out_rows(0), sem_o1).wait()

    return k(feat_flat, idx_flat, cos_flat, sin_flat)


def kernel(features, indices, freq):
    freq_rep = jnp.repeat(freq.astype(jnp.float32), 2).reshape(1, _DIM)
    cos_t, sin_t = _build_tables(freq_rep)
    return _rope_sc(features, indices.astype(jnp.int32).reshape(-1),
                    cos_t.reshape(-1), sin_t.reshape(-1))


# speedup vs baseline: 4.2579x; 1.6867x over previous
import functools

import jax
import jax.numpy as jnp
from jax import lax
from jax.experimental import pallas as pl
from jax.experimental.pallas import tpu as pltpu
from jax.experimental.pallas import tpu_sc as plsc

_DIM = 96
_GRID = 64
_D = 3 * _DIM
_N = 32768
_TB = 128
_RH = _D // 2


def _build_tables(freq2):
    def body(freq_ref, cos_ref, sin_ref):
        sm = lax.broadcasted_iota(jnp.int32, (24, 128), 1) & 63
        ang = sm.astype(jnp.float32) * freq_ref[...]
        cos_ref[...] = jnp.cos(ang)
        sin_ref[...] = jnp.sin(ang)

    return pl.pallas_call(
        body,
        out_shape=(jax.ShapeDtypeStruct((24, 128), jnp.float32),
                   jax.ShapeDtypeStruct((24, 128), jnp.float32)),
    )(freq2)


def _rope_sc(featT, idxT, cos_flat, sin_flat):
    info = plsc.get_sparse_core_info()
    nc = info.num_cores
    nw = nc * info.num_subcores
    per_w = _N // nw
    nblk = per_w // _TB
    nch = nblk * 2
    mesh = plsc.VectorSubcoreMesh(core_axis_name="c", subcore_axis_name="s")

    @functools.partial(
        pl.kernel,
        mesh=mesh,
        compiler_params=pltpu.CompilerParams(needs_layout_passes=False),
        out_type=jax.ShapeDtypeStruct((_D, _N), jnp.float32),
        scratch_types=[
            pltpu.VMEM((_RH, _TB), jnp.float32),
            pltpu.VMEM((_RH, _TB), jnp.float32),
            pltpu.VMEM((_RH, _TB), jnp.float32),
            pltpu.VMEM((_RH, _TB), jnp.float32),
            pltpu.VMEM((4, _N // 32), jnp.int32),
            pltpu.VMEM((48 * _GRID,), jnp.float32),
            pltpu.VMEM((48 * _GRID,), jnp.float32),
            pltpu.SemaphoreType.DMA,
            pltpu.SemaphoreType.DMA,
            pltpu.SemaphoreType.DMA,
            pltpu.SemaphoreType.DMA,
        ],
    )
    def k(feat_hbm, idx_hbm, cos_hbm, sin_hbm, out_hbm,
          fbuf0, fbuf1, obuf0, obuf1, cbuf, cosb, sinb,
          sem_i0, sem_i1, sem_o0, sem_o1):
        wid = lax.axis_index("s") * nc + lax.axis_index("c")
        tok0 = pl.multiple_of(wid * per_w, _TB)
        pltpu.sync_copy(cos_hbm, cosb)
        pltpu.sync_copy(sin_hbm, sinb)
        pltpu.sync_copy(idx_hbm.at[:, pl.ds(tok0, per_w)], cbuf)
        fbufs, obufs = (fbuf0, fbuf1), (obuf0, obuf1)
        sems_i, sems_o = (sem_i0, sem_i1), (sem_o0, sem_o1)

        def feat_win(b, h):
            base = pl.multiple_of(tok0 + b * _TB, _TB)
            return feat_hbm.at[pl.ds(h * _RH, _RH), pl.ds(base, _TB)]

        def out_win(b, h):
            base = pl.multiple_of(tok0 + b * _TB, _TB)
            return out_hbm.at[pl.ds(h * _RH, _RH), pl.ds(base, _TB)]

        def compute(b, h, fbuf, obuf):
            @plsc.parallel_loop(0, _TB // 16)
            def grp(g):
                col = b * _TB + g * 16
                axes = (0, 1) if h == 0 else (1, 2)
                sv = {a: cbuf[3 - a, pl.ds(col, 16)] for a in axes}
                c0 = g * 16
                for lr in range(0, _RH, 2):
                    gr = h * _RH + lr
                    a = gr // _DIM
                    kk = (gr % _DIM) // 2
                    idxv = sv[a] + kk * _GRID
                    f0 = fbuf[lr, pl.ds(c0, 16)]
                    f1 = fbuf[lr + 1, pl.ds(c0, 16)]
                    cv = plsc.load_gather(cosb, [idxv])
                    sn = plsc.load_gather(sinb, [idxv])
                    obuf[lr, pl.ds(c0, 16)] = f0 * cv - f1 * sn
                    obuf[lr + 1, pl.ds(c0, 16)] = f1 * cv + f0 * sn

        pltpu.async_copy(feat_win(0, 0), fbuf0, sem_i0)
        pltpu.async_copy(feat_win(0, 1), fbuf1, sem_i1)

        def pair(p, carry):
            for h in range(2):
                c = p * 2 + h
                pltpu.make_async_copy(feat_win(0, 0), fbufs[h], sems_i[h]).wait()

                @pl.when(p > 0)
                def _():
                    pltpu.make_async_copy(
                        obufs[h], out_win(0, 0), sems_o[h]).wait()

                compute(p, h, fbufs[h], obufs[h])
                pltpu.async_copy(obufs[h], out_win(p, h), sems_o[h])

                @pl.when(c + 2 < nch)
                def _():
                    pltpu.async_copy(feat_win(p + 1, h), fbufs[h], sems_i[h])
            return carry

        lax.fori_loop(0, nblk, pair, 0)
        pltpu.make_async_copy(obuf0, out_win(0, 0), sem_o0).wait()
        pltpu.make_async_copy(obuf1, out_win(0, 1), sem_o1).wait()

    return k(featT, idxT, cos_flat, sin_flat)


def kernel(features, indices, freq):
    freq2 = jnp.repeat(freq.astype(jnp.float32), _GRID).reshape(24, 128)
    cos_t, sin_t = _build_tables(freq2)
    outT = _rope_sc(features.T, indices.astype(jnp.int32).T,
                    cos_t.reshape(-1), sin_t.reshape(-1))
    return outT.T


# speedup vs baseline: 6.3229x; 1.4850x over previous
import functools

import jax
import jax.numpy as jnp
from jax import lax
from jax.experimental import pallas as pl
from jax.experimental.pallas import tpu as pltpu
from jax.experimental.pallas import tpu_sc as plsc

_DIM = 96
_GRID = 64
_D = 3 * _DIM
_N = 32768
_TB = 128
_RH = _D // 2


def _build_tables(freq2):
    def body(freq_ref, cos_ref, sin_ref):
        sm = lax.broadcasted_iota(jnp.int32, (24, 128), 1) & 63
        ang = sm.astype(jnp.float32) * freq_ref[...]
        cos_ref[...] = jnp.cos(ang)
        sin_ref[...] = jnp.sin(ang)

    return pl.pallas_call(
        body,
        out_shape=(jax.ShapeDtypeStruct((24, 128), jnp.float32),
                   jax.ShapeDtypeStruct((24, 128), jnp.float32)),
    )(freq2)


def _rope_sc(featT, idxT, cos_flat, sin_flat):
    info = plsc.get_sparse_core_info()
    nc = info.num_cores
    nw = nc * info.num_subcores
    per_w = _N // nw
    nblk = per_w // _TB
    nch = nblk * 2
    mesh = plsc.VectorSubcoreMesh(core_axis_name="c", subcore_axis_name="s")

    @functools.partial(
        pl.kernel,
        mesh=mesh,
        compiler_params=pltpu.CompilerParams(needs_layout_passes=False),
        out_type=jax.ShapeDtypeStruct((_D, _N), jnp.float32),
        scratch_types=[
            pltpu.VMEM((_RH, _TB), jnp.float32),
            pltpu.VMEM((_RH, _TB), jnp.float32),
            pltpu.VMEM((_RH, _TB), jnp.float32),
            pltpu.VMEM((_RH, _TB), jnp.float32),
            pltpu.VMEM((4, _N // 32), jnp.int32),
            pltpu.VMEM((48 * _GRID,), jnp.float32),
            pltpu.VMEM((48 * _GRID,), jnp.float32),
            pltpu.SemaphoreType.DMA,
            pltpu.SemaphoreType.DMA,
            pltpu.SemaphoreType.DMA,
            pltpu.SemaphoreType.DMA,
        ],
    )
    def k(feat_hbm, idx_hbm, cos_hbm, sin_hbm, out_hbm,
          fbuf0, fbuf1, obuf0, obuf1, cbuf, cosb, sinb,
          sem_i0, sem_i1, sem_o0, sem_o1):
        wid = lax.axis_index("s") * nc + lax.axis_index("c")
        tok0 = pl.multiple_of(wid * per_w, _TB)
        pltpu.sync_copy(cos_hbm, cosb)
        pltpu.sync_copy(sin_hbm, sinb)
        pltpu.sync_copy(idx_hbm.at[:, pl.ds(tok0, per_w)], cbuf)
        fbufs, obufs = (fbuf0, fbuf1), (obuf0, obuf1)
        sems_i, sems_o = (sem_i0, sem_i1), (sem_o0, sem_o1)

        def feat_win(b, h):
            base = pl.multiple_of(tok0 + b * _TB, _TB)
            return feat_hbm.at[pl.ds(h * _RH, _RH), pl.ds(base, _TB)]

        def out_win(b, h):
            base = pl.multiple_of(tok0 + b * _TB, _TB)
            return out_hbm.at[pl.ds(h * _RH, _RH), pl.ds(base, _TB)]

        def compute(b, h, fbuf, obuf):
            @plsc.parallel_loop(0, _TB // 16)
            def grp(g):
                col = b * _TB + g * 16
                axes = (0, 1) if h == 0 else (1, 2)
                sv = {a: cbuf[3 - a, pl.ds(col, 16)] for a in axes}
                c0 = g * 16
                nb = 8
                for lr0 in range(0, _RH, 2 * nb):
                    pairs = []
                    for q in range(nb):
                        lr = lr0 + 2 * q
                        gr = h * _RH + lr
                        a = gr // _DIM
                        kk = (gr % _DIM) // 2
                        pairs.append((lr, sv[a] + kk * _GRID))
                    f0s = [fbuf[lr, pl.ds(c0, 16)] for lr, _ in pairs]
                    f1s = [fbuf[lr + 1, pl.ds(c0, 16)] for lr, _ in pairs]
                    cvs = [plsc.load_gather(cosb, [ix]) for _, ix in pairs]
                    sns = [plsc.load_gather(sinb, [ix]) for _, ix in pairs]
                    for q, (lr, _) in enumerate(pairs):
                        obuf[lr, pl.ds(c0, 16)] = (
                            f0s[q] * cvs[q] - f1s[q] * sns[q])
                        obuf[lr + 1, pl.ds(c0, 16)] = (
                            f1s[q] * cvs[q] + f0s[q] * sns[q])

        pltpu.async_copy(feat_win(0, 0), fbuf0, sem_i0)
        pltpu.async_copy(feat_win(0, 1), fbuf1, sem_i1)

        def pair(p, carry):
            for h in range(2):
                c = p * 2 + h
                pltpu.make_async_copy(feat_win(0, 0), fbufs[h], sems_i[h]).wait()

                @pl.when(p > 0)
                def _():
                    pltpu.make_async_copy(
                        obufs[h], out_win(0, 0), sems_o[h]).wait()

                compute(p, h, fbufs[h], obufs[h])
                pltpu.async_copy(obufs[h], out_win(p, h), sems_o[h])

                @pl.when(c + 2 < nch)
                def _():
                    pltpu.async_copy(feat_win(p + 1, h), fbufs[h], sems_i[h])
            return carry

        lax.fori_loop(0, nblk, pair, 0)
        pltpu.make_async_copy(obuf0, out_win(0, 0), sem_o0).wait()
        pltpu.make_async_copy(obuf1, out_win(0, 1), sem_o1).wait()

    return k(featT, idxT, cos_flat, sin_flat)


def kernel(features, indices, freq):
    freq2 = jnp.repeat(freq.astype(jnp.float32), _GRID).reshape(24, 128)
    cos_t, sin_t = _build_tables(freq2)
    outT = _rope_sc(features.T, indices.astype(jnp.int32).T,
                    cos_t.reshape(-1), sin_t.reshape(-1))
    return outT.T


# speedup vs baseline: 7.5528x; 1.1945x over previous
import functools

import jax
import jax.numpy as jnp
from jax import lax
from jax.experimental import pallas as pl
from jax.experimental.pallas import tpu as pltpu
from jax.experimental.pallas import tpu_sc as plsc

_DIM = 96
_GRID = 64
_D = 3 * _DIM
_N = 32768
_TB = 128
_RH = _D // 2


def _build_tables(freq2):
    def body(freq_ref, cs_ref):
        sm = lax.broadcasted_iota(jnp.int32, (24, 128), 1) & 63
        ang = sm.astype(jnp.float32) * freq_ref[...]
        cbits = lax.bitcast_convert_type(
            jnp.cos(ang).astype(jnp.bfloat16), jnp.uint16).astype(jnp.uint32)
        sbits = lax.bitcast_convert_type(
            jnp.sin(ang).astype(jnp.bfloat16), jnp.uint16).astype(jnp.uint32)
        cs_ref[...] = ((cbits << 16) | sbits).astype(jnp.int32)

    return pl.pallas_call(
        body,
        out_shape=jax.ShapeDtypeStruct((24, 128), jnp.int32),
    )(freq2)


def _rope_sc(featT, idxT, cs_flat):
    info = plsc.get_sparse_core_info()
    nc = info.num_cores
    nw = nc * info.num_subcores
    per_w = _N // nw
    nblk = per_w // _TB
    nch = nblk * 2
    mesh = plsc.VectorSubcoreMesh(core_axis_name="c", subcore_axis_name="s")

    @functools.partial(
        pl.kernel,
        mesh=mesh,
        compiler_params=pltpu.CompilerParams(needs_layout_passes=False),
        out_type=jax.ShapeDtypeStruct((_D, _N), jnp.float32),
        scratch_types=[
            pltpu.VMEM((_RH, _TB), jnp.float32),
            pltpu.VMEM((_RH, _TB), jnp.float32),
            pltpu.VMEM((_RH, _TB), jnp.float32),
            pltpu.VMEM((_RH, _TB), jnp.float32),
            pltpu.VMEM((4, _N // 32), jnp.int32),
            pltpu.VMEM((48 * _GRID,), jnp.int32),
            pltpu.SemaphoreType.DMA,
            pltpu.SemaphoreType.DMA,
            pltpu.SemaphoreType.DMA,
            pltpu.SemaphoreType.DMA,
        ],
    )
    def k(feat_hbm, idx_hbm, cs_hbm, out_hbm,
          fbuf0, fbuf1, obuf0, obuf1, cbuf, csb,
          sem_i0, sem_i1, sem_o0, sem_o1):
        wid = lax.axis_index("s") * nc + lax.axis_index("c")
        tok0 = pl.multiple_of(wid * per_w, _TB)
        pltpu.sync_copy(cs_hbm, csb)
        pltpu.sync_copy(idx_hbm.at[:, pl.ds(tok0, per_w)], cbuf)
        fbufs, obufs = (fbuf0, fbuf1), (obuf0, obuf1)
        sems_i, sems_o = (sem_i0, sem_i1), (sem_o0, sem_o1)

        def feat_win(b, h):
            base = pl.multiple_of(tok0 + b * _TB, _TB)
            return feat_hbm.at[pl.ds(h * _RH, _RH), pl.ds(base, _TB)]

        def out_win(b, h):
            base = pl.multiple_of(tok0 + b * _TB, _TB)
            return out_hbm.at[pl.ds(h * _RH, _RH), pl.ds(base, _TB)]

        def compute(b, h, fbuf, obuf):
            @plsc.parallel_loop(0, _TB // 16)
            def grp(g):
                col = b * _TB + g * 16
                axes = (0, 1) if h == 0 else (1, 2)
                sv = {a: cbuf[3 - a, pl.ds(col, 16)] for a in axes}
                c0 = g * 16
                nb = 8
                for lr0 in range(0, _RH, 2 * nb):
                    pairs = []
                    for q in range(nb):
                        lr = lr0 + 2 * q
                        gr = h * _RH + lr
                        a = gr // _DIM
                        kk = (gr % _DIM) // 2
                        pairs.append((lr, sv[a] + kk * _GRID))
                    f0s = [fbuf[lr, pl.ds(c0, 16)] for lr, _ in pairs]
                    f1s = [fbuf[lr + 1, pl.ds(c0, 16)] for lr, _ in pairs]
                    css = [plsc.load_gather(csb, [ix]) for _, ix in pairs]
                    for q, (lr, _) in enumerate(pairs):
                        cv = plsc.bitcast(css[q] & jnp.int32(-65536),
                                          jnp.float32)
                        sn = plsc.bitcast(css[q] << 16, jnp.float32)
                        obuf[lr, pl.ds(c0, 16)] = (
                            f0s[q] * cv - f1s[q] * sn)
                        obuf[lr + 1, pl.ds(c0, 16)] = (
                            f1s[q] * cv + f0s[q] * sn)

        pltpu.async_copy(feat_win(0, 0), fbuf0, sem_i0)
        pltpu.async_copy(feat_win(0, 1), fbuf1, sem_i1)

        def pair(p, carry):
            for h in range(2):
                c = p * 2 + h
                pltpu.make_async_copy(feat_win(0, 0), fbufs[h], sems_i[h]).wait()

                @pl.when(p > 0)
                def _():
                    pltpu.make_async_copy(
                        obufs[h], out_win(0, 0), sems_o[h]).wait()

                compute(p, h, fbufs[h], obufs[h])
                pltpu.async_copy(obufs[h], out_win(p, h), sems_o[h])

                @pl.when(c + 2 < nch)
                def _():
                    pltpu.async_copy(feat_win(p + 1, h), fbufs[h], sems_i[h])
            return carry

        lax.fori_loop(0, nblk, pair, 0)
        pltpu.make_async_copy(obuf0, out_win(0, 0), sem_o0).wait()
        pltpu.make_async_copy(obuf1, out_win(0, 1), sem_o1).wait()

    return k(featT, idxT, cs_flat)


def kernel(features, indices, freq):
    freq2 = jnp.repeat(freq.astype(jnp.float32), _GRID).reshape(24, 128)
    cs_t = _build_tables(freq2)
    outT = _rope_sc(features.T, indices.astype(jnp.int32).T,
                    cs_t.reshape(-1))
    return outT.T
